# baseline (device time: 156344 ns/iter reference)
import jax
import jax.numpy as jnp
from jax import lax
from jax.experimental import pallas as pl
from jax.experimental.pallas import tpu as pltpu

N_DEV = 16
E_LOCAL = 4
N_EXP = N_DEV * E_LOCAL
T = 1024
D = 512
H = 1024
CAP_E = 48
SLAB = E_LOCAL * CAP_E


def _body(e_ref, x_ref, rw_ref, sw_ref, ew_ref, out_ref,
          disp_ref, x_recv_ref, y_send_ref, y_recv_ref,
          x_send_sems, x_recv_sems, y_send_sems, y_recv_sems):
    me = lax.axis_index("i")

    barrier_sem = pltpu.get_barrier_semaphore()
    for p in range(N_DEV):
        @pl.when(p != me)
        def _():
            pl.semaphore_signal(barrier_sem, inc=1, device_id=(p,),
                                device_id_type=pl.DeviceIdType.MESH)
    pl.semaphore_wait(barrier_sem, N_DEV - 1)

    e = e_ref[...]
    onehot_b = (e == lax.broadcasted_iota(jnp.int32, (T, N_EXP), 1))
    onehot16 = onehot_b.astype(jnp.bfloat16)
    scores = jnp.dot(x_ref[...], rw_ref[...],
                     preferred_element_type=jnp.float32)
    smax = jnp.max(scores, axis=1, keepdims=True)
    ex = jnp.exp(scores - smax)
    probs = ex / jnp.sum(ex, axis=1, keepdims=True)
    p = jnp.sum(probs * onehot_b.astype(jnp.float32), axis=1,
                keepdims=True)
    S = lax.dot_general(onehot16, onehot16, (((1,), (1,)), ((), ())),
                        preferred_element_type=jnp.float32)
    L = (lax.broadcasted_iota(jnp.int32, (T, T), 0) >
         lax.broadcasted_iota(jnp.int32, (T, T), 1))
    rank = jnp.sum(jnp.where(L, S, 0.0), axis=1,
                   keepdims=True).astype(jnp.int32)
    slot = jnp.where(rank < CAP_E, e * CAP_E + rank, -1)
    xp16 = (x_ref[...] * p).astype(jnp.bfloat16)

    for o in range(N_DEV):
        m = lax.rem(me + o, N_DEV)
        ids = lax.broadcasted_iota(jnp.int32, (T, SLAB), 1) + m * SLAB
        P_t = (slot == ids).astype(jnp.bfloat16)
        disp_ref[m] = lax.dot_general(
            P_t, xp16, (((0,), (0,)), ((), ())),
            preferred_element_type=jnp.float32).astype(jnp.bfloat16)
        if o == 0:
            cp = pltpu.make_async_copy(
                disp_ref.at[m], x_recv_ref.at[m], x_recv_sems.at[m])
            cp.start()
        else:
            rdma = pltpu.make_async_remote_copy(
                src_ref=disp_ref.at[m],
                dst_ref=x_recv_ref.at[me],
                send_sem=x_send_sems.at[m],
                recv_sem=x_recv_sems.at[me],
                device_id=(m,),
                device_id_type=pl.DeviceIdType.MESH,
            )
            rdma.start()

    out_ref[...] = jnp.dot(x_ref[...].astype(jnp.bfloat16),
                           sw_ref[...].astype(jnp.bfloat16),
                           preferred_element_type=jnp.float32)

    for o in range(N_DEV):
        s = lax.rem(me - o + N_DEV, N_DEV)
        recv = pltpu.make_async_remote_copy(
            src_ref=disp_ref.at[s],
            dst_ref=x_recv_ref.at[s],
            send_sem=x_send_sems.at[s],
            recv_sem=x_recv_sems.at[s],
            device_id=(s,),
            device_id_type=pl.DeviceIdType.MESH,
        )
        recv.wait_recv()
        for j in range(E_LOCAL):
            y_send_ref[s, pl.ds(j * CAP_E, CAP_E), :] = jnp.dot(
                x_recv_ref[s, pl.ds(j * CAP_E, CAP_E), :],
                ew_ref[j],
                preferred_element_type=jnp.float32,
            ).astype(jnp.bfloat16)
        if o == 0:
            cp = pltpu.make_async_copy(
                y_send_ref.at[s], y_recv_ref.at[s], y_recv_sems.at[s])
            cp.start()
        else:
            ret = pltpu.make_async_remote_copy(
                src_ref=y_send_ref.at[s],
                dst_ref=y_recv_ref.at[me],
                send_sem=y_send_sems.at[s],
                recv_sem=y_recv_sems.at[me],
                device_id=(s,),
                device_id_type=pl.DeviceIdType.MESH,
            )
            ret.start()

    for s in range(N_DEV):
        ret = pltpu.make_async_remote_copy(
            src_ref=y_send_ref.at[s],
            dst_ref=y_recv_ref.at[s],
            send_sem=y_send_sems.at[s],
            recv_sem=y_recv_sems.at[s],
            device_id=(s,),
            device_id_type=pl.DeviceIdType.MESH,
        )
        ret.wait_recv()

    colids = lax.broadcasted_iota(jnp.int32, (T, N_DEV * SLAB), 1)
    G = (slot == colids).astype(jnp.bfloat16)
    Y = y_recv_ref[...].reshape(N_DEV * SLAB, H)
    out_ref[...] += jnp.dot(G, Y, preferred_element_type=jnp.float32)

    for o in range(1, N_DEV):
        s = lax.rem(me + o, N_DEV)
        snd = pltpu.make_async_remote_copy(
            src_ref=disp_ref.at[s],
            dst_ref=x_recv_ref.at[s],
            send_sem=x_send_sems.at[s],
            recv_sem=x_recv_sems.at[s],
            device_id=(s,),
            device_id_type=pl.DeviceIdType.MESH,
        )
        snd.wait_send()
        snd2 = pltpu.make_async_remote_copy(
            src_ref=y_send_ref.at[s],
            dst_ref=y_recv_ref.at[s],
            send_sem=y_send_sems.at[s],
            recv_sem=y_recv_sems.at[s],
            device_id=(s,),
            device_id_type=pl.DeviceIdType.MESH,
        )
        snd2.wait_send()


def kernel(x, router_W, route_idx, expert_W, shared_W):
    e = route_idx[:, :1].astype(jnp.int32)
    ew16 = expert_W.astype(jnp.bfloat16)

    return pl.pallas_call(
        _body,
        out_shape=jax.ShapeDtypeStruct((T, H), jnp.float32),
        in_specs=[
            pl.BlockSpec(memory_space=pltpu.VMEM),
            pl.BlockSpec(memory_space=pltpu.VMEM),
            pl.BlockSpec(memory_space=pltpu.VMEM),
            pl.BlockSpec(memory_space=pltpu.VMEM),
            pl.BlockSpec(memory_space=pltpu.VMEM),
        ],
        out_specs=pl.BlockSpec(memory_space=pltpu.VMEM),
        scratch_shapes=[
            pltpu.VMEM((N_DEV, SLAB, D), jnp.bfloat16),
            pltpu.VMEM((N_DEV, SLAB, D), jnp.bfloat16),
            pltpu.VMEM((N_DEV, SLAB, H), jnp.bfloat16),
            pltpu.VMEM((N_DEV, SLAB, H), jnp.bfloat16),
            pltpu.SemaphoreType.DMA((N_DEV,)),
            pltpu.SemaphoreType.DMA((N_DEV,)),
            pltpu.SemaphoreType.DMA((N_DEV,)),
            pltpu.SemaphoreType.DMA((N_DEV,)),
        ],
        compiler_params=pltpu.CompilerParams(
            collective_id=0,
            vmem_limit_bytes=100 * 1024 * 1024,
        ),
    )(e, x, router_W, shared_W, ew16)
